# Initial kernel scaffold; baseline (speedup 1.0000x reference)
import jax
import jax.numpy as jnp
from jax.experimental import pallas as pl


def kernel(x, w0, w1):
    raise NotImplementedError("write your pallas kernel here")


# speedup vs baseline: 21.0186x; 21.0186x over previous
import functools

import jax
import jax.numpy as jnp
from jax.experimental import pallas as pl
from jax.experimental.pallas import tpu as pltpu

N = 4096
D = 512
K = 31
BM = 256
BISECT_ITERS = 32


def _fused_body(x_ref, w0_ref, w1_ref, out_ref, emb_ref):
    i = pl.program_id(0)

    @pl.when(i == 0)
    def _encode():
        h = x_ref[:] * w0_ref[:]
        h = jnp.maximum(h, 0.0)
        h = h * w1_ref[:]
        s = jnp.sum(h * h, axis=-1, keepdims=True)
        n = jnp.sqrt(s)
        emb_ref[:] = h / jnp.maximum(n, 1e-12)

    rows = emb_ref[pl.ds(i * BM, BM), :]
    a = jax.lax.dot_general(
        rows, emb_ref[:],
        dimension_numbers=(((1,), (1,)), ((), ())),
        preferred_element_type=jnp.float32,
    )

    def body(_, carry):
        lo, hi = carry
        mid = (lo + hi) * 0.5
        cnt = jnp.sum(jnp.where(a >= mid, 1.0, 0.0), axis=1, keepdims=True)
        ge = cnt >= K
        return jnp.where(ge, mid, lo), jnp.where(ge, hi, mid)

    lo0 = jnp.full((BM, 1), -1.01, jnp.float32)
    hi0 = jnp.full((BM, 1), 1.01, jnp.float32)
    lo, _ = jax.lax.fori_loop(0, BISECT_ITERS, body, (lo0, hi0))
    out_ref[:] = jnp.where((a >= lo) & (a > 0.0), a, 0.0)


@jax.jit
def kernel(x, w0, w1):
    return pl.pallas_call(
        _fused_body,
        grid=(N // BM,),
        in_specs=[
            pl.BlockSpec((N, D), lambda i: (0, 0)),
            pl.BlockSpec((1, D), lambda i: (0, 0)),
            pl.BlockSpec((1, D), lambda i: (0, 0)),
        ],
        out_specs=pl.BlockSpec((BM, N), lambda i: (i, 0)),
        out_shape=jax.ShapeDtypeStruct((N, N), jnp.float32),
        scratch_shapes=[pltpu.VMEM((N, D), jnp.float32)],
    )(x, w0.reshape(1, D), w1.reshape(1, D))
